# Initial kernel scaffold; baseline (speedup 1.0000x reference)
import jax
import jax.numpy as jnp
from jax.experimental import pallas as pl


def kernel(entropy, h_param, running_time, msg_W, msg_b, gru_Wih, gru_Whh, gru_bih, gru_bhh, dec_W, dec_b):
    raise NotImplementedError("write your pallas kernel here")


# speedup vs baseline: 4.7942x; 4.7942x over previous
import jax
import jax.numpy as jnp
from jax import lax
from jax.experimental import pallas as pl

_N = 8
_F = 8
_H = 32
_BUDGET_SECONDS = 60 * 60.0
_GUARD = 5.0


def _fused_body(scal_ref, h_param_ref, msg_W_ref, gru_Wih_ref, gru_Whh_ref,
                gru_bih_ref, gru_bhh_ref, dec_W_ref, dec_b_ref, out_ref):
    ent = scal_ref[0, 0]
    rem = jnp.maximum(_BUDGET_SECONDS - scal_ref[0, 1], 0.0)
    rem_norm = rem / _BUDGET_SECONDS

    col = lax.broadcasted_iota(jnp.int32, (_N, _F), 1)
    feat = jnp.where(col == 0, ent, jnp.where(col == 1, rem_norm, 0.0))
    h = h_param_ref[...] + feat

    msg_W = msg_W_ref[...]
    W1 = msg_W[:, :_F]
    W2 = msg_W[:, _F:]
    p1 = jnp.dot(h, W1.T, preferred_element_type=jnp.float32)
    p2 = jnp.dot(h, W2.T, preferred_element_type=jnp.float32)
    part_shared = jnp.sum(p1, axis=0, keepdims=True)
    agg = part_shared + _N * p2
    agg = agg + _N * scal_ref[0, 2:2 + _H][None, :]

    gi = jnp.dot(agg, gru_Wih_ref[...].T, preferred_element_type=jnp.float32) + gru_bih_ref[0][None, :]
    gh = jnp.dot(h, gru_Whh_ref[...].T, preferred_element_type=jnp.float32) + gru_bhh_ref[0][None, :]
    i_r, i_z, i_n = gi[:, :_F], gi[:, _F:2 * _F], gi[:, 2 * _F:]
    h_r, h_z, h_n = gh[:, :_F], gh[:, _F:2 * _F], gh[:, 2 * _F:]
    r = jax.nn.sigmoid(i_r + h_r)
    z = jax.nn.sigmoid(i_z + h_z)
    n = jnp.tanh(i_n + r * h_n)
    h_new = (1.0 - z) * n + z * h

    hm = jnp.mean(h_new, axis=0, keepdims=True)
    lr_log = jnp.sum(hm * dec_W_ref[...]) + dec_b_ref[0, 0]
    lr = jnp.clip(jnp.exp(lr_log), 0.001, 10.0)
    lr = jnp.minimum(lr, _GUARD)
    out_ref[...] = jnp.broadcast_to(lr, (1, 1))


def kernel(entropy, h_param, running_time, msg_W, msg_b, gru_Wih, gru_Whh,
           gru_bih, gru_bhh, dec_W, dec_b):
    scal = jnp.concatenate([
        jnp.float32(entropy)[None], running_time.astype(jnp.float32), msg_b.astype(jnp.float32)
    ])[None, :]

    out = pl.pallas_call(
        _fused_body,
        out_shape=jax.ShapeDtypeStruct((1, 1), jnp.float32),
    )(scal, h_param, msg_W, gru_Wih, gru_Whh,
      gru_bih[None, :], gru_bhh[None, :], dec_W, dec_b[None, :])
    return out.reshape((1,))
